# baseline (speedup 1.0000x reference)
import functools

import jax
import jax.numpy as jnp
from jax.experimental import pallas as pl
from jax.experimental.pallas import tpu as pltpu

_NUM_Q = 4
_K = 1024
_D = 64
_CCOST = 0.25
_BLK = 2048


def _rvq_kernel(z_ref, cb_ref, cbt_ref, zrec_ref, idx_ref, loss_ref):
    iota_k = jax.lax.broadcasted_iota(jnp.int32, (1, _K), 1)

    zin0 = z_ref[...]
    residual = zin0
    rec = None
    loss_acc = jnp.float32(0.0)
    n_total = jnp.float32(_BLK * _D)

    for i in range(_NUM_Q):
        W = cb_ref[i]
        W2 = W + W
        zin = residual
        zsum = jnp.sum(zin * zin, axis=1, keepdims=True)
        wnorm = jnp.sum(W * W, axis=1)[None, :]
        mm2 = jax.lax.dot_general(
            zin, W2, (((1,), (1,)), ((), ())),
            preferred_element_type=jnp.float32)
        d = (zsum + wnorm) - mm2
        dmin = jnp.min(d, axis=1, keepdims=True)
        masked = jnp.where(d == dmin, iota_k, _K)
        idx = jnp.min(masked, axis=1)
        idxt = jnp.transpose(jnp.min(masked, axis=1, keepdims=True))
        hi = idxt >> 7
        lo_b = jnp.broadcast_to(idxt & 127, (_D, zin.shape[0]))
        zqt = None
        for c in range(_K // 128):
            g = jnp.take_along_axis(
                cbt_ref[i, :, c * 128:(c + 1) * 128], lo_b, axis=1)
            zqt = g if zqt is None else jnp.where(hi == c, g, zqt)
        zq = jnp.transpose(zqt)
        zq_ste = zin + (zq - zin)
        diff = zin - zq
        loss_acc = loss_acc + (1.0 + _CCOST) * (
            jnp.sum(diff * diff) / n_total)
        residual = residual - zq_ste
        rec = zq_ste if rec is None else rec + zq_ste
        idx_ref[:, i] = idx

    zrec_ref[...] = rec
    loss_ref[...] = jnp.full((1, 1, 1), loss_acc, jnp.float32)


@jax.jit
def kernel(z, codebooks):
    n, d = z.shape
    num_blocks = n // _BLK
    grid = (num_blocks,)
    zrec, idx, loss = pl.pallas_call(
        _rvq_kernel,
        grid=grid,
        in_specs=[
            pl.BlockSpec((_BLK, d), lambda i: (i, 0)),
            pl.BlockSpec((_NUM_Q, _K, _D), lambda i: (0, 0, 0)),
            pl.BlockSpec((_NUM_Q, _D, _K), lambda i: (0, 0, 0)),
        ],
        out_specs=[
            pl.BlockSpec((_BLK, d), lambda i: (i, 0)),
            pl.BlockSpec((_BLK, _NUM_Q), lambda i: (i, 0)),
            pl.BlockSpec((1, 1, 1), lambda i: (i, 0, 0)),
        ],
        out_shape=[
            jax.ShapeDtypeStruct((n, d), jnp.float32),
            jax.ShapeDtypeStruct((n, _NUM_Q), jnp.int32),
            jax.ShapeDtypeStruct((num_blocks, 1, 1), jnp.float32),
        ],
        compiler_params=pltpu.CompilerParams(
            dimension_semantics=("parallel",),
        ),
    )(z, codebooks, jnp.transpose(codebooks, (0, 2, 1)))
    total_loss = (jnp.sum(loss) * (_BLK / n)).astype(jnp.float32)
    return zrec, idx, total_loss


# speedup vs baseline: 1.2279x; 1.2279x over previous
import functools

import jax
import jax.numpy as jnp
from jax.experimental import pallas as pl
from jax.experimental.pallas import tpu as pltpu

_NUM_Q = 4
_K = 1024
_D = 64
_CCOST = 0.25
_BLK = 2048


def _rvq_kernel(z_ref, cb_ref, zrec_ref, idx_ref, loss_ref):
    iota_k = jax.lax.broadcasted_iota(jnp.int32, (1, _K), 1)

    zin0 = z_ref[...]
    residual = zin0
    rec = None
    loss_acc = jnp.float32(0.0)
    n_total = jnp.float32(_BLK * _D)

    for i in range(_NUM_Q):
        W = cb_ref[i]
        W2 = W + W
        zin = residual
        zsum = jnp.sum(zin * zin, axis=1, keepdims=True)
        wnorm = jnp.sum(W * W, axis=1)[None, :]
        mm2 = jax.lax.dot_general(
            zin, W2, (((1,), (1,)), ((), ())),
            preferred_element_type=jnp.float32)
        d = (zsum + wnorm) - mm2
        dmin = jnp.min(d, axis=1, keepdims=True)
        masked = jnp.where(d == dmin, iota_k, _K)
        idx_col = jnp.min(masked, axis=1, keepdims=True)
        one_hot = (iota_k == idx_col).astype(jnp.float32)
        zq = jax.lax.dot_general(
            one_hot, W, (((1,), (0,)), ((), ())),
            preferred_element_type=jnp.float32)
        zq_ste = zin + (zq - zin)
        diff = zin - zq
        loss_acc = loss_acc + (1.0 + _CCOST) * (
            jnp.sum(diff * diff) / n_total)
        residual = residual - zq_ste
        rec = zq_ste if rec is None else rec + zq_ste
        idx_ref[:, i:i + 1] = idx_col

    zrec_ref[...] = rec
    loss_ref[...] = jnp.full((1, 1, 1), loss_acc, jnp.float32)


@jax.jit
def kernel(z, codebooks):
    n, d = z.shape
    num_blocks = n // _BLK
    grid = (num_blocks,)
    zrec, idx, loss = pl.pallas_call(
        _rvq_kernel,
        grid=grid,
        in_specs=[
            pl.BlockSpec((_BLK, d), lambda i: (i, 0)),
            pl.BlockSpec((_NUM_Q, _K, _D), lambda i: (0, 0, 0)),
        ],
        out_specs=[
            pl.BlockSpec((_BLK, d), lambda i: (i, 0)),
            pl.BlockSpec((_BLK, _NUM_Q), lambda i: (i, 0)),
            pl.BlockSpec((1, 1, 1), lambda i: (i, 0, 0)),
        ],
        out_shape=[
            jax.ShapeDtypeStruct((n, d), jnp.float32),
            jax.ShapeDtypeStruct((n, _NUM_Q), jnp.int32),
            jax.ShapeDtypeStruct((num_blocks, 1, 1), jnp.float32),
        ],
        compiler_params=pltpu.CompilerParams(
            dimension_semantics=("parallel",),
        ),
    )(z, codebooks)
    total_loss = (jnp.sum(loss) * (_BLK / n)).astype(jnp.float32)
    return zrec, idx, total_loss


# speedup vs baseline: 1.2756x; 1.0389x over previous
import functools

import jax
import jax.numpy as jnp
from jax.experimental import pallas as pl
from jax.experimental.pallas import tpu as pltpu

_NUM_Q = 4
_K = 1024
_D = 64
_CCOST = 0.25
_BLK = 2048


_C = 128
_NC = _K // _C


def _rvq_kernel(z_ref, cb_ref, zrec_ref, idx_ref, loss_ref):
    iota_c = jax.lax.broadcasted_iota(jnp.int32, (1, _C), 1)

    zin0 = z_ref[...]
    residual = zin0
    rec = None
    sq_acc = None
    n_total = jnp.float32(_BLK * _D)

    for i in range(_NUM_Q):
        W = cb_ref[i]
        W2 = W + W
        zin = residual
        zsum = jnp.sum(zin * zin, axis=1, keepdims=True)
        wnorm = jnp.sum(W * W, axis=1)[None, :]
        mm2 = jax.lax.dot_general(
            zin, W2, (((1,), (1,)), ((), ())),
            preferred_element_type=jnp.float32)
        d = (zsum + wnorm) - mm2
        dmin = jnp.min(d, axis=1, keepdims=True)
        fc = jnp.full((zin.shape[0], _C), _NC, jnp.int32)
        for c in reversed(range(_NC)):
            fc = jnp.where(d[:, c * _C:(c + 1) * _C] == dmin, c, fc)
        key = fc * _C + iota_c
        idx_col = jnp.min(key, axis=1, keepdims=True)
        lo_col = jnp.bitwise_and(idx_col, _C - 1)
        hi_col = jnp.right_shift(idx_col, 7)
        ohl = (iota_c == lo_col).astype(jnp.float32)
        one_hot = jnp.concatenate(
            [ohl * (hi_col == c).astype(jnp.float32)
             for c in range(_NC)], axis=1)
        zq = jax.lax.dot_general(
            one_hot, W, (((1,), (0,)), ((), ())),
            preferred_element_type=jnp.float32)
        zq_ste = zin + (zq - zin)
        residual = residual - zq_ste
        rec = zq_ste if rec is None else rec + zq_ste
        sq = jnp.sum(residual * residual, axis=1, keepdims=True)
        sq_acc = sq if sq_acc is None else sq_acc + sq
        idx_ref[:, i:i + 1] = idx_col

    zrec_ref[...] = rec
    loss_acc = (1.0 + _CCOST) * (jnp.sum(sq_acc) / n_total)
    loss_ref[...] = jnp.full((1, 1, 1), loss_acc, jnp.float32)


@jax.jit
def kernel(z, codebooks):
    n, d = z.shape
    num_blocks = n // _BLK
    grid = (num_blocks,)
    zrec, idx, loss = pl.pallas_call(
        _rvq_kernel,
        grid=grid,
        in_specs=[
            pl.BlockSpec((_BLK, d), lambda i: (i, 0)),
            pl.BlockSpec((_NUM_Q, _K, _D), lambda i: (0, 0, 0)),
        ],
        out_specs=[
            pl.BlockSpec((_BLK, d), lambda i: (i, 0)),
            pl.BlockSpec((_BLK, _NUM_Q), lambda i: (i, 0)),
            pl.BlockSpec((1, 1, 1), lambda i: (i, 0, 0)),
        ],
        out_shape=[
            jax.ShapeDtypeStruct((n, d), jnp.float32),
            jax.ShapeDtypeStruct((n, _NUM_Q), jnp.int32),
            jax.ShapeDtypeStruct((num_blocks, 1, 1), jnp.float32),
        ],
        compiler_params=pltpu.CompilerParams(
            dimension_semantics=("parallel",),
        ),
    )(z, codebooks)
    total_loss = (jnp.sum(loss) * (_BLK / n)).astype(jnp.float32)
    return zrec, idx, total_loss


# speedup vs baseline: 1.3113x; 1.0280x over previous
import functools

import jax
import jax.numpy as jnp
from jax.experimental import pallas as pl
from jax.experimental.pallas import tpu as pltpu

_NUM_Q = 4
_K = 1024
_D = 64
_CCOST = 0.25
_BLK = 2048


_C = 128
_NC = _K // _C


def _rvq_kernel(z_ref, cb_ref, zrec_ref, idx_ref, loss_ref):
    iota_c = jax.lax.broadcasted_iota(jnp.int32, (1, _C), 1)
    iota_k = jax.lax.broadcasted_iota(jnp.int32, (1, _K), 1)

    zin0 = z_ref[...]
    residual = zin0
    rec = None
    sq_acc = None
    n_total = jnp.float32(_BLK * _D)

    for i in range(_NUM_Q):
        W = cb_ref[i]
        W2 = W + W
        zin = residual
        zsum = jnp.sum(zin * zin, axis=1, keepdims=True)
        wnorm = jnp.sum(W * W, axis=1)[None, :]
        mm2 = jax.lax.dot_general(
            zin, W2, (((1,), (1,)), ((), ())),
            preferred_element_type=jnp.float32)
        d = (zsum + wnorm) - mm2
        dmin = jnp.min(d, axis=1, keepdims=True)
        fc = jnp.full((zin.shape[0], _C), _NC, jnp.int32)
        for c in reversed(range(_NC)):
            fc = jnp.where(d[:, c * _C:(c + 1) * _C] == dmin, c, fc)
        key = fc * _C + iota_c
        idx_col = jnp.min(key, axis=1, keepdims=True)
        one_hot = (iota_k == idx_col).astype(jnp.float32)
        zq = jax.lax.dot_general(
            one_hot, W, (((1,), (0,)), ((), ())),
            preferred_element_type=jnp.float32)
        zq_ste = zin + (zq - zin)
        residual = residual - zq_ste
        rec = zq_ste if rec is None else rec + zq_ste
        sq = jnp.sum(residual * residual, axis=1, keepdims=True)
        sq_acc = sq if sq_acc is None else sq_acc + sq
        idx_ref[:, i:i + 1] = idx_col

    zrec_ref[...] = rec
    loss_acc = (1.0 + _CCOST) * (jnp.sum(sq_acc) / n_total)
    loss_ref[...] = jnp.full((1, 1, 1), loss_acc, jnp.float32)


@jax.jit
def kernel(z, codebooks):
    n, d = z.shape
    num_blocks = n // _BLK
    grid = (num_blocks,)
    zrec, idx, loss = pl.pallas_call(
        _rvq_kernel,
        grid=grid,
        in_specs=[
            pl.BlockSpec((_BLK, d), lambda i: (i, 0)),
            pl.BlockSpec((_NUM_Q, _K, _D), lambda i: (0, 0, 0)),
        ],
        out_specs=[
            pl.BlockSpec((_BLK, d), lambda i: (i, 0)),
            pl.BlockSpec((_BLK, _NUM_Q), lambda i: (i, 0)),
            pl.BlockSpec((1, 1, 1), lambda i: (i, 0, 0)),
        ],
        out_shape=[
            jax.ShapeDtypeStruct((n, d), jnp.float32),
            jax.ShapeDtypeStruct((n, _NUM_Q), jnp.int32),
            jax.ShapeDtypeStruct((num_blocks, 1, 1), jnp.float32),
        ],
        compiler_params=pltpu.CompilerParams(
            dimension_semantics=("parallel",),
        ),
    )(z, codebooks)
    total_loss = (jnp.sum(loss) * (_BLK / n)).astype(jnp.float32)
    return zrec, idx, total_loss
